# baseline (speedup 1.0000x reference)
import functools

import jax
import jax.numpy as jnp
from jax import lax
from jax.experimental import pallas as pl
from jax.experimental.pallas import tpu as pltpu
from jax.experimental.pallas import tpu_sc as plsc

VOCAB = 1000000
EMBED_DIM = 64
BATCH = 16384
HIST = 50
B = BATCH * HIST
WPAD = 80

NC, NS = 2, 16
NW = NC * NS
BW = BATCH // NW
G = 128
CH = 2 * G
K = CH // G
NCHUNK = HIST * 2
NI = 4
L = 16

_mesh = plsc.VectorSubcoreMesh(core_axis_name="c", subcore_axis_name="s")


@functools.partial(
    pl.kernel,
    mesh=_mesh,
    out_type=jax.ShapeDtypeStruct(
        (HIST, EMBED_DIM // 8, BATCH // 128, 8, 128), jnp.float32
    ),
    scratch_types=[
        pltpu.VMEM((NI, CH), jnp.int32),
        pltpu.VMEM((2, CH, WPAD), jnp.float32),
        pltpu.VMEM((2, EMBED_DIM, CH), jnp.float32),
        pltpu.SemaphoreType.DMA,
        pltpu.SemaphoreType.DMA,
        pltpu.SemaphoreType.DMA,
        pltpu.SemaphoreType.DMA,
        pltpu.SemaphoreType.DMA,
        pltpu.SemaphoreType.DMA,
        pltpu.SemaphoreType.DMA,
        pltpu.SemaphoreType.DMA,
    ],
    compiler_params=pltpu.CompilerParams(
        use_tc_tiling_on_sc=False, needs_layout_passes=False
    ),
)
def _gather(table, xt, out, idx_v, rows_v, trans_v,
            si0, si1, si2, si3, sg0, sg1, so0, so1):
    wid = lax.axis_index("s") * NC + lax.axis_index("c")
    base_b = wid * BW
    si = (si0, si1, si2, si3)
    sg = (sg0, sg1)
    so = (so0, so1)

    def chunk_pos(g):
        return g // 2, base_b + (g % 2) * CH

    def idx_copy(g, a):
        h, b0 = chunk_pos(g)
        return pltpu.make_async_copy(
            xt.at[h, pl.ds(b0, CH)], idx_v.at[a], si[a]
        )

    def out_copies(g, b):
        h, b0 = chunk_pos(g)
        jg = b0 // 128
        return [
            pltpu.make_async_copy(
                trans_v.at[b].at[pl.ds(8 * d8, 8), pl.ds(128 * j, 128)],
                out.at[h, d8, jg + j],
                so[b],
            )
            for d8 in range(EMBED_DIM // 8)
            for j in range(CH // 128)
        ]

    def out_start(g, b):
        for c in out_copies(g, b):
            c.start()

    def out_wait(g, b):
        for c in out_copies(g, b):
            c.wait()

    def fire_gathers(a, b):
        for j in range(K):
            pltpu.async_copy(
                table.at[idx_v.at[a].at[pl.ds(j * G, G)]],
                rows_v.at[b].at[pl.ds(j * G, G)],
                sg[b],
            )

    def drain_gathers(b):
        for j in range(K):
            pltpu.make_async_copy(
                table.at[pl.ds(0, G)],
                rows_v.at[b].at[pl.ds(j * G, G)],
                sg[b],
            ).wait()

    _iota = jax.lax.iota(jnp.int32, L)
    _rot = [
        [jnp.asarray(q * L, jnp.int32) + (_iota + s) % L for s in range(L)]
        for q in range(EMBED_DIM // L)
    ]

    def transpose(b):
        @plsc.parallel_loop(0, CH, step=L, unroll=2)
        def bgroup(c0):
            lanev = _iota + c0
            for q in range(EMBED_DIM // L):
                for s in range(L):
                    dv = _rot[q][s]
                    vals = plsc.load_gather(rows_v.at[b], [lanev, dv])
                    plsc.store_scatter(trans_v.at[b], [dv, lanev], vals)

    def run_chunk(g, b):
        av = lax.rem(g, NI)
        for slot in range(NI):
            @pl.when(av == slot)
            def _():
                idx_copy(g, slot).wait()
                fire_gathers(slot, b)
        @pl.when(g >= 3)
        def _():
            out_wait(g - 3, b ^ 1)
        drain_gathers(b ^ 1)
        transpose(b ^ 1)
        out_start(g - 1, b ^ 1)
        pv = lax.rem(g + 3, NI)
        for slot in range(NI):
            @pl.when((pv == slot) & (g + 3 <= NCHUNK - 1))
            def _():
                idx_copy(g + 3, slot).start()

    for a in range(NI):
        idx_copy(a, a).start()
    idx_copy(0, 0).wait()
    fire_gathers(0, 0)
    idx_copy(1, 1).wait()
    fire_gathers(1, 1)
    drain_gathers(0)
    transpose(0)
    out_start(0, 0)
    idx_copy(4, 0).start()

    def pair_body(gg, carry):
        g = gg * 2
        run_chunk(g, 0)
        run_chunk(g + 1, 1)
        return carry

    lax.fori_loop(1, NCHUNK // 2, pair_body, 0)

    out_wait(97, 1)
    drain_gathers(1)
    transpose(1)
    out_start(99, 1)
    out_wait(98, 0)
    out_wait(99, 1)


def kernel(x, weight):
    xt = x.T
    wpad = jnp.pad(weight, ((0, 0), (0, WPAD - EMBED_DIM)))
    out5 = _gather(wpad, xt)
    out3 = out5.transpose(0, 1, 3, 2, 4).reshape(HIST, EMBED_DIM, BATCH)
    return jnp.transpose(out3, (2, 0, 1))


# speedup vs baseline: 1.7923x; 1.7923x over previous
import functools

import jax
import jax.numpy as jnp
from jax import lax
from jax.experimental import pallas as pl
from jax.experimental.pallas import tpu as pltpu
from jax.experimental.pallas import tpu_sc as plsc

VOCAB = 1000000
EMBED_DIM = 64
BATCH = 16384
HIST = 50
B = BATCH * HIST
WPAD = 128

NC, NS = 2, 16
NW = NC * NS
BW = BATCH // NW
G = 128
CH = 2 * G
K = CH // G
NCHUNK = HIST * 2
NI = 4
L = 16

_mesh = plsc.VectorSubcoreMesh(core_axis_name="c", subcore_axis_name="s")


@functools.partial(
    pl.kernel,
    mesh=_mesh,
    out_type=jax.ShapeDtypeStruct(
        (HIST, EMBED_DIM // 8, BATCH // 128, 8, 128), jnp.float32
    ),
    scratch_types=[
        pltpu.VMEM((NI, CH), jnp.int32),
        pltpu.VMEM((2, CH, WPAD), jnp.float32),
        pltpu.VMEM((2, EMBED_DIM, CH), jnp.float32),
        pltpu.SemaphoreType.DMA,
        pltpu.SemaphoreType.DMA,
        pltpu.SemaphoreType.DMA,
        pltpu.SemaphoreType.DMA,
        pltpu.SemaphoreType.DMA,
        pltpu.SemaphoreType.DMA,
        pltpu.SemaphoreType.DMA,
        pltpu.SemaphoreType.DMA,
    ],
    compiler_params=pltpu.CompilerParams(
        use_tc_tiling_on_sc=False, needs_layout_passes=False
    ),
)
def _gather(table, xt, out, idx_v, rows_v, trans_v,
            si0, si1, si2, si3, sg0, sg1, so0, so1):
    wid = lax.axis_index("s") * NC + lax.axis_index("c")
    base_b = wid * BW
    si = (si0, si1, si2, si3)
    sg = (sg0, sg1)
    so = (so0, so1)

    def chunk_pos(g):
        return g // 2, base_b + (g % 2) * CH

    def idx_copy(g, a):
        h, b0 = chunk_pos(g)
        return pltpu.make_async_copy(
            xt.at[h, pl.ds(b0, CH)], idx_v.at[a], si[a]
        )

    def out_copies(g, b):
        h, b0 = chunk_pos(g)
        jg = b0 // 128
        return [
            pltpu.make_async_copy(
                trans_v.at[b].at[pl.ds(8 * d8, 8), pl.ds(128 * j, 128)],
                out.at[h, d8, jg + j],
                so[b],
            )
            for d8 in range(EMBED_DIM // 8)
            for j in range(CH // 128)
        ]

    def out_start(g, b):
        for c in out_copies(g, b):
            c.start()

    def out_wait(g, b):
        for c in out_copies(g, b):
            c.wait()

    def fire_gathers(a, b):
        for j in range(K):
            pltpu.async_copy(
                table.at[idx_v.at[a].at[pl.ds(j * G, G)]],
                rows_v.at[b].at[pl.ds(j * G, G)],
                sg[b],
            )

    def drain_gathers(b):
        for j in range(K):
            pltpu.make_async_copy(
                table.at[pl.ds(0, G)],
                rows_v.at[b].at[pl.ds(j * G, G)],
                sg[b],
            ).wait()

    _iota = jax.lax.iota(jnp.int32, L)
    _rot = [
        [jnp.asarray(q * L, jnp.int32) + (_iota + s) % L for s in range(L)]
        for q in range(EMBED_DIM // L)
    ]

    def transpose(b):
        @plsc.parallel_loop(0, CH, step=L, unroll=4)
        def bgroup(c0):
            lanev = _iota + c0
            for q in range(EMBED_DIM // L):
                for s in range(L):
                    dv = _rot[q][s]
                    vals = plsc.load_gather(rows_v.at[b], [lanev, dv])
                    plsc.store_scatter(trans_v.at[b], [dv, lanev], vals)

    def run_chunk(g, b):
        av = lax.rem(g, NI)
        for slot in range(NI):
            @pl.when(av == slot)
            def _():
                idx_copy(g, slot).wait()
                fire_gathers(slot, b)
        @pl.when(g >= 3)
        def _():
            out_wait(g - 3, b ^ 1)
        drain_gathers(b ^ 1)
        transpose(b ^ 1)
        out_start(g - 1, b ^ 1)
        pv = lax.rem(g + 3, NI)
        for slot in range(NI):
            @pl.when((pv == slot) & (g + 3 <= NCHUNK - 1))
            def _():
                idx_copy(g + 3, slot).start()

    for a in range(NI):
        idx_copy(a, a).start()
    idx_copy(0, 0).wait()
    fire_gathers(0, 0)
    idx_copy(1, 1).wait()
    fire_gathers(1, 1)
    drain_gathers(0)
    transpose(0)
    out_start(0, 0)
    idx_copy(4, 0).start()

    def pair_body(gg, carry):
        g = gg * 2
        run_chunk(g, 0)
        run_chunk(g + 1, 1)
        return carry

    lax.fori_loop(1, NCHUNK // 2, pair_body, 0)

    out_wait(97, 1)
    drain_gathers(1)
    transpose(1)
    out_start(99, 1)
    out_wait(98, 0)
    out_wait(99, 1)


def kernel(x, weight):
    xt = x.T
    wpad = jnp.pad(weight, ((0, 0), (0, WPAD - EMBED_DIM)))
    out5 = _gather(wpad, xt)
    out3 = out5.transpose(0, 1, 3, 2, 4).reshape(HIST, EMBED_DIM, BATCH)
    return jnp.transpose(out3, (2, 0, 1))
